# Initial kernel scaffold; baseline (speedup 1.0000x reference)
import functools

import jax
import jax.numpy as jnp
from jax import lax
from jax.experimental import pallas as pl
from jax.experimental.pallas import tpu as pltpu
from jax.experimental.pallas import tpu_sc as plsc

_N1, _N2, _N3 = 10000, 2048, 1024
_D = 128
_NTILES = 32
_CHUNK = 128


def _make_seg_sum(n_table, n_chunks, n_keep, n_acc, clamp_to):
    mesh = plsc.VectorSubcoreMesh(core_axis_name="c", subcore_axis_name="s")
    rows_pt = n_acc // 16
    nk_pt = n_keep // 16

    @functools.partial(
        pl.kernel,
        mesh=mesh,
        out_type=(
            jax.ShapeDtypeStruct((2, n_keep, _D), jnp.float32),
            jax.ShapeDtypeStruct((2, n_keep, 16), jnp.float32),
        ),
        scratch_types=[
            pltpu.VMEM((n_chunks, _CHUNK), jnp.int32),
            pltpu.VMEM((n_chunks, _CHUNK), jnp.int32),
            pltpu.VMEM((_CHUNK, _D), jnp.float32),
            pltpu.VMEM((_CHUNK, 16), jnp.float32),
            pltpu.VMEM((64, _D), jnp.float32),
            pltpu.VMEM((64, 16), jnp.float32),
            pltpu.VMEM_SHARED((n_acc, _D), jnp.float32),
            pltpu.VMEM_SHARED((n_acc, 16), jnp.float32),
        ],
    )
    def seg(table, src3, dst3, s_out, cnt_out,
            src_idx, dst_idx, rows, ones, zrows, zcnt, acc, cacc):
        cid = lax.axis_index("c")
        sid = lax.axis_index("s")
        wid = cid * 16 + sid

        def init_const(i, carry):
            for jj in range(_D // 16):
                zrows[i, pl.ds(jj * 16, 16)] = jnp.zeros((16,), jnp.float32)
            zcnt[i, :] = jnp.zeros((16,), jnp.float32)
            return carry

        lax.fori_loop(0, 64, init_const, 0)

        def init_ones(i, carry):
            ones[i, :] = jnp.ones((16,), jnp.float32)
            return carry

        lax.fori_loop(0, _CHUNK, init_ones, 0)

        base = sid * rows_pt
        for k in range(rows_pt // 64):
            pltpu.sync_copy(zrows, acc.at[pl.ds(base + k * 64, 64)])
            pltpu.sync_copy(zcnt, cacc.at[pl.ds(base + k * 64, 64)])
        plsc.subcore_barrier()

        pltpu.sync_copy(src3.at[wid], src_idx)
        pltpu.sync_copy(dst3.at[wid], dst_idx)

        def clamp_body(t, carry):
            r = t // (_CHUNK // 16)
            c0 = (t % (_CHUNK // 16)) * 16
            dst_idx[r, pl.ds(c0, 16)] = jnp.minimum(
                dst_idx[r, pl.ds(c0, 16)], clamp_to)
            return carry

        lax.fori_loop(0, n_chunks * (_CHUNK // 16), clamp_body, 0)

        def edge_body(j, carry):
            pltpu.sync_copy(table.at[src_idx.at[j]], rows)
            pltpu.sync_copy(rows, acc.at[dst_idx.at[j]], add=True)
            pltpu.sync_copy(ones, cacc.at[dst_idx.at[j]], add=True)
            return carry

        lax.fori_loop(0, n_chunks, edge_body, 0)
        plsc.subcore_barrier()

        ob = sid * nk_pt
        pltpu.sync_copy(acc.at[pl.ds(ob, nk_pt)], s_out.at[cid, pl.ds(ob, nk_pt)])
        pltpu.sync_copy(cacc.at[pl.ds(ob, nk_pt)], cnt_out.at[cid, pl.ds(ob, nk_pt)])

    return seg


_seg1 = _make_seg_sum(50000, 79, _N2, 3072, _N2)
_seg2 = _make_seg_sum(_N2, 16, _N3, 2048, _N3)
_seg3 = _make_seg_sum(_N3, 8, _N3, 2048, _N3)


def _dense_body(s_ref, cnt_ref, x_ref, wl_ref, bl_ref, wr_ref, a_ref, o_ref):
    c = jnp.maximum(cnt_ref[0, :, 0:1] + cnt_ref[1, :, 0:1], 1.0)
    mean = (s_ref[0] + s_ref[1]) / c
    z = (jnp.dot(mean, wl_ref[...], preferred_element_type=jnp.float32)
         + jnp.dot(x_ref[...], wr_ref[...], preferred_element_type=jnp.float32)
         + bl_ref[...])
    o_ref[...] = jnp.where(z >= 0.0, z, a_ref[...] * z)


def _dense(s, cnt, x_tgt, wl, bl, wr, a):
    n = s.shape[1]
    return pl.pallas_call(
        _dense_body,
        out_shape=jax.ShapeDtypeStruct((n, _D), jnp.float32),
    )(s, cnt, x_tgt, wl.T, bl.reshape(1, _D), wr.T, a.reshape(1, _D))


def _prep_edges(edge_index, n_chunks, pad_dst):
    e = edge_index.shape[1]
    total = _NTILES * n_chunks * _CHUNK
    src = edge_index[0]
    dst = edge_index[1]
    if total > e:
        pad = total - e
        src = jnp.concatenate([src, jnp.zeros((pad,), jnp.int32)])
        dst = jnp.concatenate([dst, jnp.full((pad,), pad_dst, jnp.int32)])
    return (src.reshape(_NTILES, n_chunks, _CHUNK),
            dst.reshape(_NTILES, n_chunks, _CHUNK))


def kernel(x, edge_index1, edge_index2, edge_index3,
           Wl1, bl1, Wr1, Wl2, bl2, Wr2, Wl3, bl3, Wr3,
           a1, a2, a3):
    src1, dst1 = _prep_edges(edge_index1, 79, _N1)
    src2, dst2 = _prep_edges(edge_index2, 16, _N2)
    src3, dst3 = _prep_edges(edge_index3, 8, _N3)

    s1, c1 = _seg1(x, src1, dst1)
    h1 = _dense(s1, c1, x[:_N2], Wl1, bl1, Wr1, a1)
    s2, c2 = _seg2(h1, src2, dst2)
    h2 = _dense(s2, c2, h1[:_N3], Wl2, bl2, Wr2, a2)
    s3, c3 = _seg3(h2, src3, dst3)
    return _dense(s3, c3, h2, Wl3, bl3, Wr3, a3)


# speedup vs baseline: 5.4397x; 5.4397x over previous
import functools

import jax
import jax.numpy as jnp
from jax import lax
from jax.experimental import pallas as pl
from jax.experimental.pallas import tpu as pltpu
from jax.experimental.pallas import tpu_sc as plsc

_N1, _N2, _N3 = 10000, 2048, 1024
_D = 128
_NTILES = 32
_CHUNK = 128


def _make_seg_sum(n_table, n_chunks, n_keep, n_acc, clamp_to):
    mesh = plsc.VectorSubcoreMesh(core_axis_name="c", subcore_axis_name="s")
    rows_pt = n_acc // 16
    nk_pt = n_keep // 16

    @functools.partial(
        pl.kernel,
        mesh=mesh,
        out_type=(
            jax.ShapeDtypeStruct((2, n_keep, _D), jnp.float32),
            jax.ShapeDtypeStruct((2, n_keep, _D), jnp.float32),
        ),
        scratch_types=[
            pltpu.VMEM((n_chunks, _CHUNK), jnp.int32),
            pltpu.VMEM((n_chunks, _CHUNK), jnp.int32),
            pltpu.VMEM((_CHUNK, _D), jnp.float32),
            pltpu.VMEM((_CHUNK, _D), jnp.float32),
            pltpu.VMEM((64, _D), jnp.float32),
            pltpu.VMEM_SHARED((n_acc, _D), jnp.float32),
            pltpu.VMEM_SHARED((n_acc, _D), jnp.float32),
        ],
    )
    def seg(table, src3, dst3, s_out, cnt_out,
            src_idx, dst_idx, rows, ones, zrows, acc, cacc):
        cid = lax.axis_index("c")
        sid = lax.axis_index("s")
        wid = cid * 16 + sid

        def init_const(i, carry):
            for jj in range(_D // 16):
                zrows[i, pl.ds(jj * 16, 16)] = jnp.zeros((16,), jnp.float32)
            return carry

        lax.fori_loop(0, 64, init_const, 0)

        def init_ones(i, carry):
            for jj in range(_D // 16):
                ones[i, pl.ds(jj * 16, 16)] = jnp.ones((16,), jnp.float32)
            return carry

        lax.fori_loop(0, _CHUNK, init_ones, 0)

        base = sid * rows_pt
        for k in range(rows_pt // 64):
            pltpu.sync_copy(zrows, acc.at[pl.ds(base + k * 64, 64)])
            pltpu.sync_copy(zrows, cacc.at[pl.ds(base + k * 64, 64)])
        plsc.subcore_barrier()

        pltpu.sync_copy(src3.at[wid], src_idx)
        pltpu.sync_copy(dst3.at[wid], dst_idx)

        def clamp_body(t, carry):
            r = t // (_CHUNK // 16)
            c0 = (t % (_CHUNK // 16)) * 16
            dst_idx[r, pl.ds(c0, 16)] = jnp.minimum(
                dst_idx[r, pl.ds(c0, 16)], clamp_to)
            return carry

        lax.fori_loop(0, n_chunks * (_CHUNK // 16), clamp_body, 0)

        def edge_body(j, carry):
            pltpu.sync_copy(table.at[src_idx.at[j]], rows)
            pltpu.sync_copy(rows, acc.at[dst_idx.at[j]], add=True)
            pltpu.sync_copy(ones, cacc.at[dst_idx.at[j]], add=True)
            return carry

        lax.fori_loop(0, n_chunks, edge_body, 0)
        plsc.subcore_barrier()

        ob = sid * nk_pt
        pltpu.sync_copy(acc.at[pl.ds(ob, nk_pt)], s_out.at[cid, pl.ds(ob, nk_pt)])
        pltpu.sync_copy(cacc.at[pl.ds(ob, nk_pt)], cnt_out.at[cid, pl.ds(ob, nk_pt)])

    return seg


_seg1 = _make_seg_sum(50000, 79, _N2, 3072, _N2)
_seg2 = _make_seg_sum(_N2, 16, _N3, 2048, _N3)
_seg3 = _make_seg_sum(_N3, 8, _N3, 2048, _N3)


def _dense_body(s_ref, cnt_ref, x_ref, wl_ref, bl_ref, wr_ref, a_ref, o_ref):
    c = jnp.maximum(cnt_ref[0, :, 0:1] + cnt_ref[1, :, 0:1], 1.0)
    mean = (s_ref[0] + s_ref[1]) / c
    z = (jnp.dot(mean, wl_ref[...], preferred_element_type=jnp.float32)
         + jnp.dot(x_ref[...], wr_ref[...], preferred_element_type=jnp.float32)
         + bl_ref[...])
    o_ref[...] = jnp.where(z >= 0.0, z, a_ref[...] * z)


def _dense(s, cnt, x_tgt, wl, bl, wr, a):
    n = s.shape[1]
    return pl.pallas_call(
        _dense_body,
        out_shape=jax.ShapeDtypeStruct((n, _D), jnp.float32),
    )(s, cnt, x_tgt, wl.T, bl.reshape(1, _D), wr.T, a.reshape(1, _D))


def _prep_edges(edge_index, n_chunks, pad_dst):
    e = edge_index.shape[1]
    total = _NTILES * n_chunks * _CHUNK
    src = edge_index[0]
    dst = edge_index[1]
    if total > e:
        pad = total - e
        src = jnp.concatenate([src, jnp.zeros((pad,), jnp.int32)])
        dst = jnp.concatenate([dst, jnp.full((pad,), pad_dst, jnp.int32)])
    return (src.reshape(_NTILES, n_chunks, _CHUNK),
            dst.reshape(_NTILES, n_chunks, _CHUNK))


def kernel(x, edge_index1, edge_index2, edge_index3,
           Wl1, bl1, Wr1, Wl2, bl2, Wr2, Wl3, bl3, Wr3,
           a1, a2, a3):
    src1, dst1 = _prep_edges(edge_index1, 79, _N1)
    src2, dst2 = _prep_edges(edge_index2, 16, _N2)
    src3, dst3 = _prep_edges(edge_index3, 8, _N3)

    s1, c1 = _seg1(x, src1, dst1)
    h1 = _dense(s1, c1, x[:_N2], Wl1, bl1, Wr1, a1)
    s2, c2 = _seg2(h1, src2, dst2)
    h2 = _dense(s2, c2, h1[:_N3], Wl2, bl2, Wr2, a2)
    s3, c3 = _seg3(h2, src3, dst3)
    return _dense(s3, c3, h2, Wl3, bl3, Wr3, a3)


# speedup vs baseline: 9.3196x; 1.7132x over previous
import functools

import jax
import jax.numpy as jnp
from jax import lax
from jax.experimental import pallas as pl
from jax.experimental.pallas import tpu as pltpu
from jax.experimental.pallas import tpu_sc as plsc

_N1, _N2, _N3 = 10000, 2048, 1024
_D = 128
_NTILES = 32
_CHUNK = 128


def _make_seg_sum(n_table, n_chunks, n_keep, n_acc, clamp_to):
    mesh = plsc.VectorSubcoreMesh(core_axis_name="c", subcore_axis_name="s")
    rows_pt = n_acc // 16
    nk_pt = n_keep // 16

    @functools.partial(
        pl.kernel,
        mesh=mesh,
        out_type=(
            jax.ShapeDtypeStruct((2, n_keep, _D), jnp.float32),
            jax.ShapeDtypeStruct((2, n_keep, _D), jnp.float32),
        ),
        scratch_types=[
            pltpu.VMEM((n_chunks, _CHUNK), jnp.int32),
            pltpu.VMEM((n_chunks, _CHUNK), jnp.int32),
            pltpu.VMEM(((n_chunks + 1) * _CHUNK,), jnp.int32),
            pltpu.VMEM(((n_chunks + 1) * _CHUNK,), jnp.int32),
            pltpu.VMEM((_CHUNK,), jnp.int32),
            pltpu.VMEM((_CHUNK, _D), jnp.float32),
            pltpu.VMEM((_CHUNK, _D), jnp.float32),
            pltpu.VMEM((32, _D), jnp.float32),
            pltpu.VMEM_SHARED((n_acc, _D), jnp.float32),
            pltpu.VMEM_SHARED((n_acc, _D), jnp.float32),
        ],
        compiler_params=pltpu.CompilerParams(needs_layout_passes=False),
    )
    def seg(table, src3, dst3, s_out, cnt_out,
            src_idx, dst_idx, csrc, cdst, cdrow, rows, ones, zrows, acc, cacc):
        cid = lax.axis_index("c")
        sid = lax.axis_index("s")
        wid = cid * 16 + sid

        def init_const(i, carry):
            for jj in range(_D // 16):
                zrows[i, pl.ds(jj * 16, 16)] = jnp.zeros((16,), jnp.float32)
            return carry

        lax.fori_loop(0, 32, init_const, 0)

        def init_ones(i, carry):
            for jj in range(_D // 16):
                ones[i, pl.ds(jj * 16, 16)] = jnp.ones((16,), jnp.float32)
            return carry

        lax.fori_loop(0, _CHUNK, init_ones, 0)

        base = sid * rows_pt
        for k in range(rows_pt // 32):
            pltpu.sync_copy(zrows, acc.at[pl.ds(base + k * 32, 32)])
            pltpu.sync_copy(zrows, cacc.at[pl.ds(base + k * 32, 32)])
        plsc.subcore_barrier()

        pltpu.sync_copy(src3.at[wid], src_idx)
        pltpu.sync_copy(dst3.at[wid], dst_idx)

        def comp_body(t, wp):
            r = t // (_CHUNK // 16)
            c0 = (t % (_CHUNK // 16)) * 16
            vd = dst_idx[r, pl.ds(c0, 16)]
            vs = src_idx[r, pl.ds(c0, 16)]
            kd, ks = plsc.sort_key_val(vd, vs)
            cdst[pl.ds(wp, 16)] = kd
            csrc[pl.ds(wp, 16)] = ks
            return wp + jnp.sum((kd < clamp_to).astype(jnp.int32))

        wp = lax.fori_loop(0, n_chunks * (_CHUNK // 16), comp_body,
                           jnp.int32(0))
        for t in range(_CHUNK // 16):
            cdst[pl.ds(wp + t * 16, 16)] = jnp.full((16,), clamp_to, jnp.int32)
            csrc[pl.ds(wp + t * 16, 16)] = jnp.zeros((16,), jnp.int32)
        n_live = (wp + _CHUNK - 1) // _CHUNK

        def edge_body(j, carry):
            pltpu.sync_copy(table.at[csrc.at[pl.ds(j * _CHUNK, _CHUNK)]], rows)
            for k in range(_CHUNK // 16):
                cdrow[pl.ds(k * 16, 16)] = cdst[pl.ds(j * _CHUNK + k * 16, 16)]
            pltpu.sync_copy(rows, acc.at[cdrow], add=True)
            pltpu.sync_copy(ones, cacc.at[cdrow], add=True)
            return carry

        lax.fori_loop(0, n_live, edge_body, 0)
        plsc.subcore_barrier()

        ob = sid * nk_pt
        pltpu.sync_copy(acc.at[pl.ds(ob, nk_pt)], s_out.at[cid, pl.ds(ob, nk_pt)])
        pltpu.sync_copy(cacc.at[pl.ds(ob, nk_pt)], cnt_out.at[cid, pl.ds(ob, nk_pt)])

    return seg


_seg1 = _make_seg_sum(50000, 79, _N2, 2560, _N2)
_seg2 = _make_seg_sum(_N2, 16, _N3, 1536, _N3)
_seg3 = _make_seg_sum(_N3, 8, _N3, 1536, _N3)


def _dense_body(s_ref, cnt_ref, x_ref, wl_ref, bl_ref, wr_ref, a_ref, o_ref):
    c = jnp.maximum(cnt_ref[0, :, 0:1] + cnt_ref[1, :, 0:1], 1.0)
    mean = (s_ref[0] + s_ref[1]) / c
    z = (jnp.dot(mean, wl_ref[...], preferred_element_type=jnp.float32)
         + jnp.dot(x_ref[...], wr_ref[...], preferred_element_type=jnp.float32)
         + bl_ref[...])
    o_ref[...] = jnp.where(z >= 0.0, z, a_ref[...] * z)


def _dense(s, cnt, x_tgt, wl, bl, wr, a):
    n = s.shape[1]
    return pl.pallas_call(
        _dense_body,
        out_shape=jax.ShapeDtypeStruct((n, _D), jnp.float32),
    )(s, cnt, x_tgt, wl.T, bl.reshape(1, _D), wr.T, a.reshape(1, _D))


def _prep_edges(edge_index, n_chunks, pad_dst):
    e = edge_index.shape[1]
    total = _NTILES * n_chunks * _CHUNK
    src = edge_index[0]
    dst = edge_index[1]
    if total > e:
        pad = total - e
        src = jnp.concatenate([src, jnp.zeros((pad,), jnp.int32)])
        dst = jnp.concatenate([dst, jnp.full((pad,), pad_dst, jnp.int32)])
    return (src.reshape(_NTILES, n_chunks, _CHUNK),
            dst.reshape(_NTILES, n_chunks, _CHUNK))


def kernel(x, edge_index1, edge_index2, edge_index3,
           Wl1, bl1, Wr1, Wl2, bl2, Wr2, Wl3, bl3, Wr3,
           a1, a2, a3):
    src1, dst1 = _prep_edges(edge_index1, 79, _N1)
    src2, dst2 = _prep_edges(edge_index2, 16, _N2)
    src3, dst3 = _prep_edges(edge_index3, 8, _N3)

    s1, c1 = _seg1(x, src1, dst1)
    h1 = _dense(s1, c1, x[:_N2], Wl1, bl1, Wr1, a1)
    s2, c2 = _seg2(h1, src2, dst2)
    h2 = _dense(s2, c2, h1[:_N3], Wl2, bl2, Wr2, a2)
    s3, c3 = _seg3(h2, src3, dst3)
    return _dense(s3, c3, h2, Wl3, bl3, Wr3, a3)
